# Initial kernel scaffold; baseline (speedup 1.0000x reference)
import jax
import jax.numpy as jnp
from jax.experimental import pallas as pl


def kernel(x, y, v, s, W1, b1, W2, b2):
    raise NotImplementedError("write your pallas kernel here")


# speedup vs baseline: 2.1577x; 2.1577x over previous
import functools

import jax
import jax.numpy as jnp
from jax import lax
from jax.experimental import pallas as pl
from jax.experimental.pallas import tpu as pltpu
from jax.experimental.pallas import tpu_sc as plsc

HIDDEN = 1024
DOUT = 1030
S_SIZE = 6
B = 1024
F = 32
L2_IN = 2 * HIDDEN
PSQT_SCALE = 32.0 / 361.0

DPAD = 1032
NC, NS, LANES = 2, 16, 16
NW = NC * NS
BPW = B // NW
NCHUNK = HIDDEN // LANES
TAIL = DOUT - LANES


def _accumulate(rows, va, vb, b1_buf, o):
    def cstep(c, carry):
        off = c * LANES
        acc = b1_buf[pl.ds(off, LANES)]
        for f in range(LANES):
            acc = acc + va[f] * rows[f, pl.ds(off, LANES)]
        for f in range(LANES):
            acc = acc + vb[f] * rows[LANES + f, pl.ds(off, LANES)]
        o[pl.ds(off, LANES)] = acc
        return carry

    lax.fori_loop(0, NCHUNK, cstep, 0, unroll=False)
    acc = b1_buf[pl.ds(TAIL, LANES)]
    for f in range(LANES):
        acc = acc + va[f] * rows[f, pl.ds(TAIL, LANES)]
    for f in range(LANES):
        acc = acc + vb[f] * rows[LANES + f, pl.ds(TAIL, LANES)]
    o[pl.ds(TAIL, LANES)] = acc


def _sc_body(xy_hbm, v_hbm, s_hbm, w1_hbm, b1_hbm, w2_hbm, b2_hbm, out_hbm,
             idx_buf, v_buf, s_buf, rows_a, rows_b, w2row, b1_buf, b2_buf,
             xo, yo, cpart, out_v, sem_a, sem_b, sem_w2):
    wid = lax.axis_index("s") * NC + lax.axis_index("c")
    base_b = wid * BPW
    base_t = wid * (2 * BPW)

    pltpu.sync_copy(xy_hbm.at[pl.ds(base_t, 2 * BPW)], idx_buf)
    pltpu.sync_copy(v_hbm.at[pl.ds(base_b * F, BPW * F)], v_buf)
    pltpu.sync_copy(s_hbm.at[pl.ds(base_b, BPW)], s_buf)
    pltpu.sync_copy(b1_hbm, b1_buf)
    pltpu.sync_copy(b2_hbm, b2_buf)

    pltpu.async_copy(w1_hbm.at[idx_buf.at[0]], rows_a, sem_a)

    lane = jnp.arange(LANES, dtype=jnp.int32)

    def bstep(b, carry):
        t0 = 2 * b
        b16 = jnp.full((LANES,), b, dtype=jnp.int32)
        s16 = plsc.load_gather(s_buf, [b16])
        s_b = s16[0]

        pltpu.async_copy(w1_hbm.at[idx_buf.at[t0 + 1]], rows_b, sem_b)
        pltpu.async_copy(w2_hbm.at[pl.ds(s_b, 1)], w2row, sem_w2)

        va = v_buf[pl.ds(b * F, LANES)]
        vb = v_buf[pl.ds(b * F + LANES, LANES)]

        pltpu.make_async_copy(w1_hbm.at[idx_buf.at[t0]], rows_a, sem_a).wait()
        _accumulate(rows_a, va, vb, b1_buf, xo)

        @pl.when(b < BPW - 1)
        def _():
            pltpu.async_copy(w1_hbm.at[idx_buf.at[t0 + 2]], rows_a, sem_a)

        pltpu.make_async_copy(w1_hbm.at[idx_buf.at[t0 + 1]], rows_b, sem_b).wait()
        _accumulate(rows_b, va, vb, b1_buf, yo)

        pltpu.make_async_copy(w2_hbm.at[pl.ds(s_b, 1)], w2row, sem_w2).wait()

        def l2step(c, acc):
            off = c * LANES
            xr = jnp.maximum(xo[pl.ds(off, LANES)], 0.0)
            yr = jnp.maximum(yo[pl.ds(off, LANES)], 0.0)
            return (acc + xr * w2row[0, pl.ds(off, LANES)]
                    + yr * w2row[0, pl.ds(HIDDEN + off, LANES)])

        acc = lax.fori_loop(0, NCHUNK, l2step,
                            jnp.zeros((LANES,), jnp.float32), unroll=False)
        b2v = plsc.load_gather(b2_buf, [s16])
        xg = plsc.load_gather(xo, [HIDDEN + s16])
        yg = plsc.load_gather(yo, [HIDDEN + s16])
        extra = b2v + (xg - yg) * PSQT_SCALE
        acc = acc + jnp.where(lane == 0, extra, 0.0)
        cpart[b, pl.ds(0, LANES)] = acc
        return carry

    lax.fori_loop(0, BPW, bstep, 0, unroll=False)

    for g in range(BPW // LANES):
        rowids = g * LANES + lane
        tot = jnp.zeros((LANES,), jnp.float32)
        for k in range(LANES):
            tot = tot + plsc.load_gather(
                cpart, [rowids, jnp.full((LANES,), k, dtype=jnp.int32)])
        out_v[pl.ds(g * LANES, LANES)] = 1.0 / (1.0 + jnp.exp(-tot))
    pltpu.sync_copy(out_v, out_hbm.at[pl.ds(base_b, BPW)])


_sc_forward = functools.partial(
    pl.kernel,
    out_type=jax.ShapeDtypeStruct((B,), jnp.float32),
    mesh=plsc.VectorSubcoreMesh(core_axis_name="c", subcore_axis_name="s",
                                num_cores=NC, num_subcores=NS),
    compiler_params=pltpu.CompilerParams(needs_layout_passes=False,
                                         use_tc_tiling_on_sc=False),
    scratch_types=[
        pltpu.VMEM((2 * BPW, F), jnp.int32),
        pltpu.VMEM((BPW * F,), jnp.float32),
        pltpu.VMEM((BPW,), jnp.int32),
        pltpu.VMEM((F, DPAD), jnp.float32),
        pltpu.VMEM((F, DPAD), jnp.float32),
        pltpu.VMEM((1, L2_IN), jnp.float32),
        pltpu.VMEM((DOUT,), jnp.float32),
        pltpu.VMEM((LANES,), jnp.float32),
        pltpu.VMEM((DOUT,), jnp.float32),
        pltpu.VMEM((DOUT,), jnp.float32),
        pltpu.VMEM((BPW, LANES), jnp.float32),
        pltpu.VMEM((BPW,), jnp.float32),
        pltpu.SemaphoreType.DMA,
        pltpu.SemaphoreType.DMA,
        pltpu.SemaphoreType.DMA,
    ],
)(_sc_body)


def kernel(x, y, v, s, W1, b1, W2, b2):
    xy = jnp.stack([x, y], axis=1).reshape(2 * B, F)
    b2p = jnp.pad(b2, (0, LANES - S_SIZE))
    w1p = jnp.pad(W1, ((0, 0), (0, DPAD - DOUT)))
    out = _sc_forward(xy, v.reshape(B * F), s.reshape(B), w1p, b1, W2, b2p)
    return out.reshape(B, 1)


# speedup vs baseline: 2.1605x; 1.0013x over previous
import functools

import jax
import jax.numpy as jnp
from jax import lax
from jax.experimental import pallas as pl
from jax.experimental.pallas import tpu as pltpu
from jax.experimental.pallas import tpu_sc as plsc

HIDDEN = 1024
DOUT = 1030
S_SIZE = 6
B = 1024
F = 32
L2_IN = 2 * HIDDEN
TPAD = 128
PSQT_SCALE = 32.0 / 361.0

NC, NS, LANES = 2, 16, 16
NW = NC * NS
BPW = B // NW
NCHUNK = HIDDEN // LANES


def _accumulate(rows, trows, va, vb, b1h_buf, b1t_buf, o, ops):
    def cstep(c, carry):
        off = c * LANES
        acc = b1h_buf[pl.ds(off, LANES)]
        for f in range(LANES):
            acc = acc + va[f] * rows[f, pl.ds(off, LANES)]
        for f in range(LANES):
            acc = acc + vb[f] * rows[LANES + f, pl.ds(off, LANES)]
        o[pl.ds(off, LANES)] = acc
        return carry

    lax.fori_loop(0, NCHUNK, cstep, 0, unroll=False)
    acc = b1t_buf[pl.ds(0, LANES)]
    for f in range(LANES):
        acc = acc + va[f] * trows[f, pl.ds(0, LANES)]
    for f in range(LANES):
        acc = acc + vb[f] * trows[LANES + f, pl.ds(0, LANES)]
    ops[pl.ds(0, LANES)] = acc


def _sc_body(xy_hbm, v_hbm, s_hbm, w1_hbm, w1t_hbm, b1h_hbm, b1t_hbm, w2_hbm,
             b2_hbm, out_hbm,
             idx_buf, v_buf, s_buf, rows_a, rows_b, trows_a, trows_b, w2all,
             b1h_buf, b1t_buf, b2_buf, xo, yo, xps, yps, cpart, out_v,
             sem_a, sem_b, sem_ta, sem_tb):
    wid = lax.axis_index("s") * NC + lax.axis_index("c")
    base_b = wid * BPW
    base_t = wid * (2 * BPW)

    pltpu.sync_copy(xy_hbm.at[pl.ds(base_t, 2 * BPW)], idx_buf)
    pltpu.sync_copy(v_hbm.at[pl.ds(base_b * F, BPW * F)], v_buf)
    pltpu.sync_copy(s_hbm.at[pl.ds(base_b, BPW)], s_buf)
    pltpu.sync_copy(b1h_hbm, b1h_buf)
    pltpu.sync_copy(b1t_hbm, b1t_buf)
    pltpu.sync_copy(b2_hbm, b2_buf)
    pltpu.sync_copy(w2_hbm, w2all)

    lane = jnp.arange(LANES, dtype=jnp.int32)

    pltpu.async_copy(w1_hbm.at[idx_buf.at[0], pl.ds(0, HIDDEN)], rows_a, sem_a)
    pltpu.async_copy(w1t_hbm.at[idx_buf.at[0]], trows_a, sem_ta)
    pltpu.async_copy(w1_hbm.at[idx_buf.at[1], pl.ds(0, HIDDEN)], rows_b, sem_b)
    pltpu.async_copy(w1t_hbm.at[idx_buf.at[1]], trows_b, sem_tb)

    def bstep(b, carry):
        t0 = 2 * b
        b16 = jnp.full((LANES,), b, dtype=jnp.int32)
        s16 = plsc.load_gather(s_buf, [b16])

        va = v_buf[pl.ds(b * F, LANES)]
        vb = v_buf[pl.ds(b * F + LANES, LANES)]

        pltpu.make_async_copy(w1_hbm.at[idx_buf.at[t0], pl.ds(0, HIDDEN)],
                              rows_a, sem_a).wait()
        pltpu.make_async_copy(w1t_hbm.at[idx_buf.at[t0]], trows_a, sem_ta).wait()
        _accumulate(rows_a, trows_a, va, vb, b1h_buf, b1t_buf, xo, xps)

        @pl.when(b < BPW - 1)
        def _():
            pltpu.async_copy(w1_hbm.at[idx_buf.at[t0 + 2], pl.ds(0, HIDDEN)],
                             rows_a, sem_a)
            pltpu.async_copy(w1t_hbm.at[idx_buf.at[t0 + 2]], trows_a, sem_ta)

        pltpu.make_async_copy(w1_hbm.at[idx_buf.at[t0 + 1], pl.ds(0, HIDDEN)],
                              rows_b, sem_b).wait()
        pltpu.make_async_copy(w1t_hbm.at[idx_buf.at[t0 + 1]], trows_b,
                              sem_tb).wait()
        _accumulate(rows_b, trows_b, va, vb, b1h_buf, b1t_buf, yo, yps)

        @pl.when(b < BPW - 1)
        def _():
            pltpu.async_copy(w1_hbm.at[idx_buf.at[t0 + 3], pl.ds(0, HIDDEN)],
                             rows_b, sem_b)
            pltpu.async_copy(w1t_hbm.at[idx_buf.at[t0 + 3]], trows_b, sem_tb)

        def l2step(c, acc):
            off = c * LANES
            col = off + lane
            xr = jnp.maximum(xo[pl.ds(off, LANES)], 0.0)
            yr = jnp.maximum(yo[pl.ds(off, LANES)], 0.0)
            wx = plsc.load_gather(w2all, [s16, col])
            wy = plsc.load_gather(w2all, [s16, HIDDEN + col])
            return acc + xr * wx + yr * wy

        acc = lax.fori_loop(0, NCHUNK, l2step,
                            jnp.zeros((LANES,), jnp.float32), unroll=False)
        b2v = plsc.load_gather(b2_buf, [s16])
        xg = plsc.load_gather(xps, [s16])
        yg = plsc.load_gather(yps, [s16])
        extra = b2v + (xg - yg) * PSQT_SCALE
        acc = acc + jnp.where(lane == 0, extra, 0.0)
        cpart[b, pl.ds(0, LANES)] = acc
        return carry

    lax.fori_loop(0, BPW, bstep, 0, unroll=False)

    for g in range(BPW // LANES):
        rowids = g * LANES + lane
        tot = jnp.zeros((LANES,), jnp.float32)
        for k in range(LANES):
            tot = tot + plsc.load_gather(
                cpart, [rowids, jnp.full((LANES,), k, dtype=jnp.int32)])
        out_v[pl.ds(g * LANES, LANES)] = 1.0 / (1.0 + jnp.exp(-tot))
    pltpu.sync_copy(out_v, out_hbm.at[pl.ds(base_b, BPW)])


_sc_forward = functools.partial(
    pl.kernel,
    out_type=jax.ShapeDtypeStruct((B,), jnp.float32),
    mesh=plsc.VectorSubcoreMesh(core_axis_name="c", subcore_axis_name="s",
                                num_cores=NC, num_subcores=NS),
    compiler_params=pltpu.CompilerParams(needs_layout_passes=False),
    scratch_types=[
        pltpu.VMEM((2 * BPW, F), jnp.int32),
        pltpu.VMEM((BPW * F,), jnp.float32),
        pltpu.VMEM((BPW,), jnp.int32),
        pltpu.VMEM((F, HIDDEN), jnp.float32),
        pltpu.VMEM((F, HIDDEN), jnp.float32),
        pltpu.VMEM((F, TPAD), jnp.float32),
        pltpu.VMEM((F, TPAD), jnp.float32),
        pltpu.VMEM((S_SIZE, L2_IN), jnp.float32),
        pltpu.VMEM((HIDDEN,), jnp.float32),
        pltpu.VMEM((LANES,), jnp.float32),
        pltpu.VMEM((LANES,), jnp.float32),
        pltpu.VMEM((HIDDEN,), jnp.float32),
        pltpu.VMEM((HIDDEN,), jnp.float32),
        pltpu.VMEM((LANES,), jnp.float32),
        pltpu.VMEM((LANES,), jnp.float32),
        pltpu.VMEM((BPW, LANES), jnp.float32),
        pltpu.VMEM((BPW,), jnp.float32),
        pltpu.SemaphoreType.DMA,
        pltpu.SemaphoreType.DMA,
        pltpu.SemaphoreType.DMA,
        pltpu.SemaphoreType.DMA,
    ],
)(_sc_body)


def kernel(x, y, v, s, W1, b1, W2, b2):
    xy = jnp.stack([x, y], axis=1).reshape(2 * B, F)
    w1t = jnp.pad(W1[:, HIDDEN:], ((0, 0), (0, TPAD - S_SIZE)))
    b1h = b1[:HIDDEN]
    b1t = jnp.pad(b1[HIDDEN:], (0, LANES - S_SIZE))
    b2p = jnp.pad(b2, (0, LANES - S_SIZE))
    out = _sc_forward(xy, v.reshape(B * F), s.reshape(B), W1, w1t, b1h,
                      b1t, W2, b2p)
    return out.reshape(B, 1)
